# baseline (speedup 1.0000x reference)
import functools

import jax
import jax.numpy as jnp
from jax.experimental import pallas as pl
from jax.experimental.pallas import tpu as pltpu
from jax.experimental.pallas import tpu_sc as plsc

VOCAB = 1000000
EMBED = 64
NFEAT = 26
HIDDEN = 1024
NCLASS = 79
BATCH = 16384
TOTAL = BATCH * NFEAT

GATHER_WINDOW = 128
BB = 1024


def _gather_rows(emb, idx_flat):
    mesh = plsc.VectorSubcoreMesh(core_axis_name="core",
                                  subcore_axis_name="subcore")

    @functools.partial(
        pl.kernel,
        out_type=jax.ShapeDtypeStruct((TOTAL, EMBED), jnp.float32),
        mesh=mesh,
        compiler_params=pltpu.CompilerParams(use_tc_tiling_on_sc=False),
    )
    def gather_kernel(emb_hbm, idx_hbm, out_hbm):
        def body(i_vmem, o_vmem):
            pltpu.sync_copy(emb_hbm.at[i_vmem.at[0]], o_vmem)

        pltpu.emit_pipeline(
            body,
            grid=(TOTAL // GATHER_WINDOW,),
            in_specs=[pl.BlockSpec((1, GATHER_WINDOW),
                                   index_map=lambda i: (0, i))],
            out_specs=[pl.BlockSpec((GATHER_WINDOW, EMBED),
                                    index_map=lambda i: (i, 0))],
            core_axis_name=("core", "subcore"),
            dimension_semantics=(pltpu.PARALLEL,),
        )(idx_hbm, out_hbm)

    return gather_kernel(emb, idx_flat)


def _mlp_kernel(x_ref, w1_ref, b1_ref, w2_ref, b2_ref, out_ref):
    h = jax.lax.dot_general(x_ref[...], w1_ref[...], (((1,), (1,)), ((), ())),
                            preferred_element_type=jnp.float32)
    h = jnp.maximum(h + b1_ref[...], 0.0)
    o = jax.lax.dot_general(h, w2_ref[...], (((1,), (1,)), ((), ())),
                            preferred_element_type=jnp.float32)
    out_ref[...] = o + b2_ref[...]


def _mlp(x, W1, b1, W2, b2):
    return pl.pallas_call(
        _mlp_kernel,
        grid=(BATCH // BB,),
        in_specs=[
            pl.BlockSpec((BB, NFEAT * EMBED), lambda i: (i, 0)),
            pl.BlockSpec((HIDDEN, NFEAT * EMBED), lambda i: (0, 0)),
            pl.BlockSpec((1, HIDDEN), lambda i: (0, 0)),
            pl.BlockSpec((NCLASS, HIDDEN), lambda i: (0, 0)),
            pl.BlockSpec((1, NCLASS), lambda i: (0, 0)),
        ],
        out_specs=pl.BlockSpec((BB, NCLASS), lambda i: (i, 0)),
        out_shape=jax.ShapeDtypeStruct((BATCH, NCLASS), jnp.float32),
    )(x, W1, b1.reshape(1, HIDDEN), W2, b2.reshape(1, NCLASS))


def kernel(t, emb, W1, b1, W2, b2):
    idx_flat = t.reshape(1, TOTAL).astype(jnp.int32)
    rows = _gather_rows(emb, idx_flat)
    x = rows.reshape(BATCH, NFEAT * EMBED)
    return _mlp(x, W1, b1, W2, b2)


# speedup vs baseline: 1.0017x; 1.0017x over previous
import functools

import jax
import jax.numpy as jnp
from jax.experimental import pallas as pl
from jax.experimental.pallas import tpu as pltpu
from jax.experimental.pallas import tpu_sc as plsc

VOCAB = 1000000
EMBED = 64
NFEAT = 26
HIDDEN = 1024
NCLASS = 79
BATCH = 16384

GATHER_WINDOW = 128
NB = BATCH // GATHER_WINDOW
BB = 1024


def _gather_rows(emb, idx):
    mesh = plsc.VectorSubcoreMesh(core_axis_name="core",
                                  subcore_axis_name="subcore")

    @functools.partial(
        pl.kernel,
        out_type=jax.ShapeDtypeStruct((BATCH, NFEAT * EMBED), jnp.float32),
        mesh=mesh,
        compiler_params=pltpu.CompilerParams(use_tc_tiling_on_sc=False),
    )
    def gather_kernel(emb_hbm, idx_hbm, out_hbm):
        def body(i_vmem, o_vmem):
            pltpu.sync_copy(emb_hbm.at[i_vmem.at[0, 0]], o_vmem)

        pltpu.emit_pipeline(
            body,
            grid=(NB, NFEAT),
            in_specs=[pl.BlockSpec((1, 1, GATHER_WINDOW),
                                   index_map=lambda b, f: (f * NB + b, 0, 0))],
            out_specs=[pl.BlockSpec((GATHER_WINDOW, EMBED),
                                    index_map=lambda b, f: (b, f))],
            core_axis_name=("core", "subcore"),
            dimension_semantics=(pltpu.PARALLEL, pltpu.PARALLEL),
        )(idx_hbm, out_hbm)

    return gather_kernel(emb, idx)


def _mlp_kernel(x_ref, w1_ref, b1_ref, w2_ref, b2_ref, out_ref):
    h = jax.lax.dot_general(x_ref[...], w1_ref[...], (((1,), (1,)), ((), ())),
                            preferred_element_type=jnp.float32)
    h = jnp.maximum(h + b1_ref[...], 0.0)
    o = jax.lax.dot_general(h, w2_ref[...], (((1,), (1,)), ((), ())),
                            preferred_element_type=jnp.float32)
    out_ref[...] = o + b2_ref[...]


def _mlp(x, W1, b1, W2, b2):
    return pl.pallas_call(
        _mlp_kernel,
        grid=(BATCH // BB,),
        in_specs=[
            pl.BlockSpec((BB, NFEAT * EMBED), lambda i: (i, 0)),
            pl.BlockSpec((HIDDEN, NFEAT * EMBED), lambda i: (0, 0)),
            pl.BlockSpec((1, HIDDEN), lambda i: (0, 0)),
            pl.BlockSpec((NCLASS, HIDDEN), lambda i: (0, 0)),
            pl.BlockSpec((1, NCLASS), lambda i: (0, 0)),
        ],
        out_specs=pl.BlockSpec((BB, NCLASS), lambda i: (i, 0)),
        out_shape=jax.ShapeDtypeStruct((BATCH, NCLASS), jnp.float32),
    )(x, W1, b1.reshape(1, HIDDEN), W2, b2.reshape(1, NCLASS))


def kernel(t, emb, W1, b1, W2, b2):
    idx = t.astype(jnp.int32).T.reshape(NFEAT * NB, 1, GATHER_WINDOW)
    x = _gather_rows(emb, idx)
    return _mlp(x, W1, b1, W2, b2)
